# Initial kernel scaffold; baseline (speedup 1.0000x reference)
import jax
import jax.numpy as jnp
from jax.experimental import pallas as pl


def kernel(status, est_size, edge_index, edge_feat1, edge_feat2, ef1_table, ef2_table, aw1, ab1, aw2, ab2, conv1_w, conv1_b, conv2_w, conv2_b, actor_w1, actor_b1, actor_w2, actor_b2, crit_w1, crit_b1, crit_w2, crit_b2, crit_w3, crit_b3):
    raise NotImplementedError("write your pallas kernel here")


# speedup vs baseline: 23.9152x; 23.9152x over previous
import functools

import jax
import jax.numpy as jnp
from jax import lax
from jax.experimental import pallas as pl
from jax.experimental.pallas import tpu as pltpu
from jax.experimental.pallas import tpu_sc as plsc

N = 50000
E = 800000
HID = 64
MAX_ID = 10000

NC = 2
NS = 16
NPT = N // NS

BN = 5000
BE = 8000

F32 = jnp.float32


def _sc_mesh():
    return plsc.VectorSubcoreMesh(
        core_axis_name="c", subcore_axis_name="s", num_cores=NC,
        num_subcores=NS)


_K1_G, _K1_NB, _K1_C = 125, 5, 40


def _k1_emb_gather(ef1p, f1r):
    @functools.partial(
        pl.kernel, mesh=_sc_mesh(),
        compiler_params=pltpu.CompilerParams(use_tc_tiling_on_sc=False),
        out_type=jax.ShapeDtypeStruct((32 * 625, _K1_C, 8), F32),
        scratch_types=[
            pltpu.VMEM((_K1_NB, _K1_C), jnp.int32),
            pltpu.VMEM((_K1_NB, _K1_C, 8), F32),
            pltpu.SemaphoreType.DMA((_K1_NB,)),
            pltpu.SemaphoreType.DMA((_K1_NB,)),
        ],
    )
    def k(tab, idx, out, ibuf, rows, gsem, osem):
        c = lax.axis_index("c")
        s = lax.axis_index("s")
        wid = s * NC + c

        def grp(g, _):
            base = wid * 625 + g * _K1_NB
            pltpu.sync_copy(idx.at[pl.ds(base, _K1_NB)], ibuf)
            for b in range(_K1_NB):
                pltpu.async_copy(tab.at[ibuf.at[b]], rows.at[b], gsem.at[b])
            for b in range(_K1_NB):
                pltpu.make_async_copy(tab.at[ibuf.at[b]], rows.at[b],
                                      gsem.at[b]).wait()
                pltpu.async_copy(rows.at[b], out.at[base + b], osem.at[b])
            for b in range(_K1_NB):
                pltpu.make_async_copy(rows.at[b], out.at[base + b],
                                      osem.at[b]).wait()
            return _

        lax.fori_loop(0, _K1_G, grp, None)

    return k(ef1p, f1r)


_KA_G, _KA_NB, _KA_C = 125, 5, 80


def _ka_gather_scatter(table, gidx, sidx, zeros, W):
    @functools.partial(
        pl.kernel, mesh=_sc_mesh(),
        compiler_params=pltpu.CompilerParams(use_tc_tiling_on_sc=False),
        out_type=jax.ShapeDtypeStruct((2 * N, W), F32),
        scratch_types=[
            pltpu.VMEM((_KA_NB, _KA_C), jnp.int32),
            pltpu.VMEM((_KA_NB, _KA_C), jnp.int32),
            pltpu.VMEM((_KA_NB, _KA_C, W), F32),
            pltpu.VMEM_SHARED((N, W), F32),
            pltpu.SemaphoreType.DMA((_KA_NB,)),
            pltpu.SemaphoreType.DMA((_KA_NB,)),
        ],
    )
    def k(tab, gi, si, z, out, gbuf, sbuf, rows, acc, gsem, ssem):
        c = lax.axis_index("c")
        s = lax.axis_index("s")
        pltpu.sync_copy(z.at[pl.ds(s * NPT, NPT)], acc.at[pl.ds(s * NPT, NPT)])
        plsc.subcore_barrier()

        def grp(g, _):
            base = c * 10000 + s * 625 + g * _KA_NB
            pltpu.sync_copy(gi.at[pl.ds(base, _KA_NB)], gbuf)
            pltpu.sync_copy(si.at[pl.ds(base, _KA_NB)], sbuf)
            for b in range(_KA_NB):
                pltpu.async_copy(tab.at[gbuf.at[b]], rows.at[b], gsem.at[b])
            for b in range(_KA_NB):
                pltpu.make_async_copy(tab.at[gbuf.at[b]], rows.at[b],
                                      gsem.at[b]).wait()
                pltpu.async_copy(rows.at[b], acc.at[sbuf.at[b]], ssem.at[b],
                                 add=True)
            for b in range(_KA_NB):
                pltpu.make_async_copy(rows.at[b], acc.at[sbuf.at[b]],
                                      ssem.at[b]).wait()
            return _

        lax.fori_loop(0, _KA_G, grp, None)
        plsc.subcore_barrier()
        pltpu.sync_copy(acc.at[pl.ds(s * NPT, NPT)],
                        out.at[pl.ds(c * N + s * NPT, NPT)])

    return k(table, gidx, sidx, zeros)


_KB_G, _KB_NB, _KB_C = 125, 5, 40


def _kb_attn_scatter(rows3, sidx_a, sidx_b, zeros):
    @functools.partial(
        pl.kernel, mesh=_sc_mesh(),
        compiler_params=pltpu.CompilerParams(use_tc_tiling_on_sc=False),
        out_type=jax.ShapeDtypeStruct((2 * N, 16), F32),
        scratch_types=[
            pltpu.VMEM((_KB_NB, _KB_C), jnp.int32),
            pltpu.VMEM((_KB_NB, _KB_C), jnp.int32),
            pltpu.VMEM((_KB_NB, _KB_C, 16), F32),
            pltpu.VMEM_SHARED((N, 16), F32),
            pltpu.SemaphoreType.DMA((_KB_NB,)),
            pltpu.SemaphoreType.DMA((_KB_NB,)),
            pltpu.SemaphoreType.DMA((_KB_NB,)),
        ],
    )
    def k(rws, sia, sib, z, out, abuf, bbuf, rows, acc, lsem, asem, bsem):
        c = lax.axis_index("c")
        s = lax.axis_index("s")
        pltpu.sync_copy(z.at[pl.ds(s * NPT, NPT)], acc.at[pl.ds(s * NPT, NPT)])
        plsc.subcore_barrier()

        def grp(g, _):
            base = c * 10000 + s * 625 + g * _KB_NB
            pltpu.sync_copy(sia.at[pl.ds(base, _KB_NB)], abuf)
            pltpu.sync_copy(sib.at[pl.ds(base, _KB_NB)], bbuf)
            pltpu.async_copy(rws.at[pl.ds(base, _KB_NB)], rows, lsem.at[0])
            pltpu.make_async_copy(rws.at[pl.ds(base, _KB_NB)], rows,
                                  lsem.at[0]).wait()
            for b in range(_KB_NB):
                pltpu.async_copy(rows.at[b], acc.at[abuf.at[b]], asem.at[b],
                                 add=True)
            for b in range(_KB_NB):
                pltpu.make_async_copy(rows.at[b], acc.at[abuf.at[b]],
                                      asem.at[b]).wait()
                pltpu.async_copy(rows.at[b], acc.at[bbuf.at[b]], bsem.at[b],
                                 add=True)
            for b in range(_KB_NB):
                pltpu.make_async_copy(rows.at[b], acc.at[bbuf.at[b]],
                                      bsem.at[b]).wait()
            return _

        lax.fori_loop(0, _KB_G, grp, None)
        plsc.subcore_barrier()
        pltpu.sync_copy(acc.at[pl.ds(s * NPT, NPT)],
                        out.at[pl.ds(c * N + s * NPT, NPT)])

    return k(rows3, sidx_a, sidx_b, zeros)


def _tc_nodeinfo(status, est_size):
    g = N // BN

    def body(st_ref, es_ref, ni_ref, ep_ref):
        st = st_ref[0, 0]
        es = es_ref[0, 0]
        one = jnp.ones((BN, 1), F32)
        cols = [one]
        for v in (-1, 0, 1):
            cols.append((st == v).astype(F32)[:, None])
        cols.append(jnp.zeros((BN, 4), F32))
        ni_ref[...] = jnp.concatenate(cols, axis=1)
        p = jnp.concatenate(
            [jnp.sum(es)[None], jnp.sum(es * es)[None], jnp.zeros((6,), F32)])
        ep_ref[...] = p.reshape(1, 1, 8)

    return pl.pallas_call(
        body,
        grid=(g,),
        compiler_params=pltpu.CompilerParams(
            vmem_limit_bytes=60 * 1024 * 1024),
        in_specs=[
            pl.BlockSpec((1, 1, BN), lambda i: (i, 0, 0)),
            pl.BlockSpec((1, 1, BN), lambda i: (i, 0, 0)),
        ],
        out_specs=[
            pl.BlockSpec((BN, 8), lambda i: (i, 0)),
            pl.BlockSpec((1, 1, 8), lambda i: (i, 0, 0)),
        ],
        out_shape=[
            jax.ShapeDtypeStruct((N, 8), F32),
            jax.ShapeDtypeStruct((g, 1, 8), F32),
        ],
    )(status, est_size)


def _tc_scores(emb1, ef2, f2, aw1, ab1, aw2, ab2):
    g = E // BE

    def body(e1_ref, f2_ref, t2_ref, w1_ref, b1_ref, w2_ref, b2_ref, o_ref):
        e1 = e1_ref[...]
        f2v = f2_ref[0, 0]
        oh = (f2v[:, None] == lax.broadcasted_iota(jnp.int32, (1, 3), 1)
              ).astype(F32)
        e2 = jnp.dot(oh, t2_ref[...], preferred_element_type=F32)
        emb8 = jnp.concatenate([e1[:, :5], e2], axis=1)
        h = jnp.tanh(jnp.dot(emb8, w1_ref[...], preferred_element_type=F32)
                     + b1_ref[...][None, :])
        sc = jnp.dot(h, w2_ref[...], preferred_element_type=F32) \
            + b2_ref[...][None, :]
        w = jnp.exp(sc)
        o_ref[...] = jnp.concatenate(
            [w, w * emb8, jnp.zeros((BE, 7), F32)], axis=1)

    return pl.pallas_call(
        body,
        grid=(g,),
        compiler_params=pltpu.CompilerParams(
            vmem_limit_bytes=60 * 1024 * 1024),
        in_specs=[
            pl.BlockSpec((BE, 8), lambda i: (i, 0)),
            pl.BlockSpec((1, 1, BE), lambda i: (i, 0, 0)),
            pl.BlockSpec((3, 3), lambda i: (0, 0)),
            pl.BlockSpec((8, 8), lambda i: (0, 0)),
            pl.BlockSpec((8,), lambda i: (0,)),
            pl.BlockSpec((8, 1), lambda i: (0, 0)),
            pl.BlockSpec((1,), lambda i: (0,)),
        ],
        out_specs=pl.BlockSpec((BE, 16), lambda i: (i, 0)),
        out_shape=jax.ShapeDtypeStruct((E, 16), F32),
    )(emb1, f2, ef2, aw1, ab1, aw2, ab2)


def _tc_feats(acc1, acc2, est, estp, conv1_w):
    g = N // BN
    gp = N // BN

    def body(a1a_ref, a1b_ref, a2a_ref, a2b_ref, es_ref, ep_ref, w_ref,
             hp_ref, dis_ref):
        ep = ep_ref[...]
        s1 = jnp.sum(ep[:, 0, 0])
        s2 = jnp.sum(ep[:, 0, 1])
        mean = s1 / N
        var = (s2 - N * mean * mean) / (N - 1)
        std = jnp.sqrt(jnp.maximum(var, 0.0))
        es = es_ref[0, 0] - mean
        es = jnp.where(std > 1e-8, es / jnp.where(std > 1e-8, std, 1.0), es)

        a1a = a1a_ref[...]
        a1b = a1b_ref[...]
        out_deg = a1a[:, 0]
        in_deg = a1b[:, 0]
        total = in_deg + out_deg
        nst = a1a[:, 1:4] + a1b[:, 1:4]
        a2 = a2a_ref[...] + a2b_ref[...]
        denom = a2[:, 0]
        vec = a2[:, 1:9]
        agg = vec / jnp.where(denom > 0, denom, 1.0)[:, None]
        feats = jnp.concatenate(
            [es[:, None], total[:, None], nst, agg], axis=1)
        h1 = jnp.dot(feats, w_ref[...], preferred_element_type=F32)
        dis = lax.rsqrt(in_deg + 1.0)
        hp = dis[:, None] * h1
        hp_ref[0] = hp[:, :32]
        hp_ref[1] = hp[:, 32:]
        dis_ref[0, 0] = dis

    return pl.pallas_call(
        body,
        grid=(g,),
        compiler_params=pltpu.CompilerParams(
            vmem_limit_bytes=60 * 1024 * 1024),
        in_specs=[
            pl.BlockSpec((BN, 8), lambda i: (i, 0)),
            pl.BlockSpec((BN, 8), lambda i: (i, 0)),
            pl.BlockSpec((BN, 16), lambda i: (i, 0)),
            pl.BlockSpec((BN, 16), lambda i: (i, 0)),
            pl.BlockSpec((1, 1, BN), lambda i: (i, 0, 0)),
            pl.BlockSpec((gp, 1, 8), lambda i: (0, 0, 0)),
            pl.BlockSpec((13, HID), lambda i: (0, 0)),
        ],
        out_specs=[
            pl.BlockSpec((2, BN, 32), lambda i: (0, i, 0)),
            pl.BlockSpec((1, 1, BN), lambda i: (i, 0, 0)),
        ],
        out_shape=[
            jax.ShapeDtypeStruct((2, N, 32), F32),
            jax.ShapeDtypeStruct((N // BN, 1, BN), F32),
        ],
    )(acc1[0], acc1[1], acc2[0], acc2[1], est, estp, conv1_w)


def _tc_layer(S, hp, dis, b_in, w_next):
    g = N // BN

    def body(sa_ref, hpa_ref, dis_ref, b_ref, w_ref, o_ref):
        sa = sa_ref[...]
        hpa = hpa_ref[...]
        dis = dis_ref[0, 0][:, None]
        pre = jnp.concatenate(
            [dis * (sa[0] + hpa[0]), dis * (sa[1] + hpa[1])], axis=1)
        x = jnp.maximum(pre + b_ref[...][None, :], 0.0)
        h = jnp.dot(x, w_ref[...], preferred_element_type=F32)
        hp = dis * h
        o_ref[0] = hp[:, :32]
        o_ref[1] = hp[:, 32:]

    return pl.pallas_call(
        body,
        grid=(g,),
        compiler_params=pltpu.CompilerParams(
            vmem_limit_bytes=60 * 1024 * 1024),
        in_specs=[
            pl.BlockSpec((2, BN, 32), lambda i: (0, i, 0)),
            pl.BlockSpec((2, BN, 32), lambda i: (0, i, 0)),
            pl.BlockSpec((1, 1, BN), lambda i: (i, 0, 0)),
            pl.BlockSpec((HID,), lambda i: (0,)),
            pl.BlockSpec((HID, HID), lambda i: (0, 0)),
        ],
        out_specs=pl.BlockSpec((2, BN, 32), lambda i: (0, i, 0)),
        out_shape=jax.ShapeDtypeStruct((2, N, 32), F32),
    )(S, hp, dis, b_in, w_next)


def _tc_x2(S, hp, dis, b_in, status):
    g = N // BN

    def body(sa_ref, hpa_ref, dis_ref, b_ref, st_ref, x_ref, p_ref):
        sa = sa_ref[...]
        hpa = hpa_ref[...]
        dis = dis_ref[0, 0][:, None]
        pre = jnp.concatenate(
            [dis * (sa[0] + hpa[0]), dis * (sa[1] + hpa[1])], axis=1)
        x = jnp.maximum(pre + b_ref[...][None, :], 0.0)
        x_ref[...] = x
        st = st_ref[0, 0]
        cnt1 = jnp.sum((st == 1).astype(F32))
        cnt0 = jnp.sum((st == 0).astype(F32))
        p = jnp.concatenate(
            [jnp.sum(x, axis=0), cnt1[None], cnt0[None],
             jnp.zeros((62,), F32)])
        p_ref[...] = p.reshape(1, 1, 128)

    return pl.pallas_call(
        body,
        grid=(g,),
        compiler_params=pltpu.CompilerParams(
            vmem_limit_bytes=60 * 1024 * 1024),
        in_specs=[
            pl.BlockSpec((2, BN, 32), lambda i: (0, i, 0)),
            pl.BlockSpec((2, BN, 32), lambda i: (0, i, 0)),
            pl.BlockSpec((1, 1, BN), lambda i: (i, 0, 0)),
            pl.BlockSpec((HID,), lambda i: (0,)),
            pl.BlockSpec((1, 1, BN), lambda i: (i, 0, 0)),
        ],
        out_specs=[
            pl.BlockSpec((BN, HID), lambda i: (i, 0)),
            pl.BlockSpec((1, 1, 128), lambda i: (i, 0, 0)),
        ],
        out_shape=[
            jax.ShapeDtypeStruct((N, HID), F32),
            jax.ShapeDtypeStruct((g, 1, 128), F32),
        ],
    )(S, hp, dis, b_in, status)


def _tc_heads(x2, partials, status, aw1, ab1, aw2, ab2):
    g = N // BN
    gp = N // BN

    def body(x_ref, p_ref, st_ref, aw1_ref, ab1_ref, aw2_ref, ab2_ref,
             lg_ref):
        p = jnp.sum(p_ref[...][:, 0, :], axis=0)
        gmean = (p[:64] / N).reshape(1, 64)
        cnt1 = p[64]
        x = x_ref[...]
        aw = aw1_ref[...]
        gterm = jnp.dot(gmean, aw[64:, :], preferred_element_type=F32) \
            + ab1_ref[...][None, :]
        h = jnp.maximum(
            jnp.dot(x, aw[:64, :], preferred_element_type=F32) + gterm, 0.0)
        raw = (jnp.dot(h, aw2_ref[...], preferred_element_type=F32)
               + ab2_ref[...][None, :])[:, 0]
        st = st_ref[0, 0]
        m1 = (st == 1).astype(F32)
        m0 = (st == 0).astype(F32)
        m = jnp.where(cnt1 > 0, m1, m0)
        lg_ref[0, 0] = raw * m + (1.0 - m) * (-1e9)

    return pl.pallas_call(
        body,
        grid=(g,),
        compiler_params=pltpu.CompilerParams(
            vmem_limit_bytes=60 * 1024 * 1024),
        in_specs=[
            pl.BlockSpec((BN, HID), lambda i: (i, 0)),
            pl.BlockSpec((gp, 1, 128), lambda i: (0, 0, 0)),
            pl.BlockSpec((1, 1, BN), lambda i: (i, 0, 0)),
            pl.BlockSpec((2 * HID, HID), lambda i: (0, 0)),
            pl.BlockSpec((HID,), lambda i: (0,)),
            pl.BlockSpec((HID, 1), lambda i: (0, 0)),
            pl.BlockSpec((1,), lambda i: (0,)),
        ],
        out_specs=pl.BlockSpec((1, 1, BN), lambda i: (i, 0, 0)),
        out_shape=jax.ShapeDtypeStruct((N // BN, 1, BN), F32),
    )(x2, partials, status, aw1, ab1, aw2, ab2)


def kernel(status, est_size, edge_index, edge_feat1, edge_feat2, ef1_table,
           ef2_table, aw1, ab1, aw2, ab2, conv1_w, conv1_b, conv2_w, conv2_b,
           actor_w1, actor_b1, actor_w2, actor_b2,
           crit_w1, crit_b1, crit_w2, crit_b2, crit_w3, crit_b3):
    src = edge_index[0]
    dst = edge_index[1]
    st2 = status.astype(jnp.int32).reshape(N // BN, 1, BN)
    es2 = est_size.astype(F32).reshape(N // BN, 1, BN)

    nodeinfo, estp = _tc_nodeinfo(st2, es2)

    ef1p = jnp.pad(ef1_table.astype(F32), ((0, 0), (0, 3)))
    f1r = edge_feat1.astype(jnp.int32).reshape(32 * 625, _K1_C)
    emb1 = _k1_emb_gather(ef1p, f1r).reshape(E, 8)

    wemb = _tc_scores(emb1, ef2_table.astype(F32),
                      edge_feat2.astype(jnp.int32).reshape(E // BE, 1, BE),
                      aw1, ab1, aw2, ab2)

    src80 = src.astype(jnp.int32).reshape(10000, 80)
    dst80 = dst.astype(jnp.int32).reshape(10000, 80)
    z8 = jnp.zeros((N, 8), F32)
    acc1 = _ka_gather_scatter(
        nodeinfo,
        jnp.concatenate([dst80, src80], 0),
        jnp.concatenate([src80, dst80], 0),
        z8, 8).reshape(2, N, 8)

    src40 = src.astype(jnp.int32).reshape(20000, _KB_C)
    dst40 = dst.astype(jnp.int32).reshape(20000, _KB_C)
    z16 = jnp.zeros((N, 16), F32)
    acc2 = _kb_attn_scatter(
        wemb.reshape(20000, _KB_C, 16), src40, dst40, z16).reshape(2, N, 16)

    h1p, dis2 = _tc_feats(acc1, acc2, es2, estp, conv1_w)

    z32 = jnp.zeros((N, 32), F32)
    gidx5 = jnp.concatenate([src80, src80 + N], 0)
    sidx5 = jnp.concatenate([dst80, dst80], 0)
    S1 = _ka_gather_scatter(h1p.reshape(2 * N, 32), gidx5, sidx5, z32,
                            32).reshape(2, N, 32)

    h2p = _tc_layer(S1, h1p, dis2, conv1_b, conv2_w)

    S2 = _ka_gather_scatter(h2p.reshape(2 * N, 32), gidx5, sidx5, z32,
                            32).reshape(2, N, 32)

    x2, partials = _tc_x2(S2, h2p, dis2, conv2_b, st2)

    logits = _tc_heads(x2, partials, st2,
                       actor_w1, actor_b1, actor_w2, actor_b2)

    gm = jnp.mean(x2, axis=0)
    combined = jnp.concatenate(
        [x2, jnp.broadcast_to(gm, (N, gm.shape[0]))], axis=1)
    pooled = jnp.mean(combined, axis=0)
    hv = jax.nn.relu(pooled @ crit_w1 + crit_b1)
    hv = jax.nn.relu(hv @ crit_w2 + crit_b2)
    value = (hv @ crit_w3 + crit_b3)[0]
    return (logits.reshape(N), value)


# speedup vs baseline: 26.9983x; 1.1289x over previous
import functools

import jax
import jax.numpy as jnp
from jax import lax
from jax.experimental import pallas as pl
from jax.experimental.pallas import tpu as pltpu
from jax.experimental.pallas import tpu_sc as plsc

N = 50000
E = 800000
HID = 64
MAX_ID = 10000

NC = 2
NS = 16
NPT = N // NS

BN = 5000
BE = 8000

F32 = jnp.float32


def _sc_mesh():
    return plsc.VectorSubcoreMesh(
        core_axis_name="c", subcore_axis_name="s", num_cores=NC,
        num_subcores=NS)


_K1_G, _K1_NB, _K1_C = 125, 5, 40


def _k1_emb_gather(ef1p, f1r):
    @functools.partial(
        pl.kernel, mesh=_sc_mesh(),
        compiler_params=pltpu.CompilerParams(use_tc_tiling_on_sc=False),
        out_type=jax.ShapeDtypeStruct((32 * 625, _K1_C, 8), F32),
        scratch_types=[
            pltpu.VMEM((_K1_NB, _K1_C), jnp.int32),
            pltpu.VMEM((_K1_NB, _K1_C, 8), F32),
            pltpu.SemaphoreType.DMA((_K1_NB,)),
            pltpu.SemaphoreType.DMA((_K1_NB,)),
        ],
    )
    def k(tab, idx, out, ibuf, rows, gsem, osem):
        c = lax.axis_index("c")
        s = lax.axis_index("s")
        wid = s * NC + c

        def grp(g, _):
            base = wid * 625 + g * _K1_NB
            pltpu.sync_copy(idx.at[pl.ds(base, _K1_NB)], ibuf)
            for b in range(_K1_NB):
                pltpu.async_copy(tab.at[ibuf.at[b]], rows.at[b], gsem.at[b])
            for b in range(_K1_NB):
                pltpu.make_async_copy(tab.at[ibuf.at[b]], rows.at[b],
                                      gsem.at[b]).wait()
                pltpu.async_copy(rows.at[b], out.at[base + b], osem.at[b])
            for b in range(_K1_NB):
                pltpu.make_async_copy(rows.at[b], out.at[base + b],
                                      osem.at[b]).wait()
            return _

        lax.fori_loop(0, _K1_G, grp, None)

    return k(ef1p, f1r)


_KA_G, _KA_NB, _KA_C = 125, 5, 80


def _ka_gather_scatter(table, idxc, zeros, W):
    G, NB, C = _KA_G, _KA_NB, _KA_C

    @functools.partial(
        pl.kernel, mesh=_sc_mesh(),
        compiler_params=pltpu.CompilerParams(use_tc_tiling_on_sc=False),
        out_type=jax.ShapeDtypeStruct((2 * N, W), F32),
        scratch_types=[
            pltpu.VMEM((3 * 2 * NB, C), jnp.int32),
            pltpu.VMEM((2 * NB, C, W), F32),
            pltpu.VMEM_SHARED((N, W), F32),
            pltpu.SemaphoreType.DMA((3,)),
            pltpu.SemaphoreType.DMA((2 * NB,)),
            pltpu.SemaphoreType.DMA((2 * NB,)),
        ],
    )
    def k(tab, idx, z, out, ibuf, rows, k_acc, isem, gsem, ssem):
        c = lax.axis_index("c")
        s = lax.axis_index("s")
        pltpu.sync_copy(z.at[pl.ds(s * NPT, NPT)], k_acc.at[pl.ds(s * NPT, NPT)])
        plsc.subcore_barrier()
        blk0 = (c * NS + s) * G

        def start_gathers(p, sl):
            for b in range(NB):
                pltpu.async_copy(tab.at[ibuf.at[sl * 2 * NB + b]],
                                 rows.at[p * NB + b], gsem.at[p * NB + b])

        def drain_gather_start_scatter(p, sl):
            for b in range(NB):
                pltpu.make_async_copy(tab.at[ibuf.at[sl * 2 * NB + b]],
                                      rows.at[p * NB + b],
                                      gsem.at[p * NB + b]).wait()
                pltpu.async_copy(rows.at[p * NB + b],
                                 k_acc.at[ibuf.at[sl * 2 * NB + NB + b]],
                                 ssem.at[p * NB + b], add=True)

        def wait_scatters(p, sl):
            for b in range(NB):
                pltpu.make_async_copy(rows.at[p * NB + b],
                                      k_acc.at[ibuf.at[sl * 2 * NB + NB + b]],
                                      ssem.at[p * NB + b]).wait()

        pltpu.sync_copy(idx.at[blk0], ibuf.at[pl.ds(0, 2 * NB)])
        start_gathers(0, 0)
        pltpu.async_copy(idx.at[blk0 + 1], ibuf.at[pl.ds(2 * NB, 2 * NB)],
                         isem.at[1])

        def grp(g, _):
            p = lax.rem(g, 2)
            q = 1 - p
            sl = lax.rem(g, 3)
            slp = lax.rem(g + 2, 3)
            slpp = lax.rem(g + 1, 3)
            drain_gather_start_scatter(q, slp)
            pltpu.make_async_copy(idx.at[blk0 + g],
                                  ibuf.at[pl.ds(sl * 2 * NB, 2 * NB)],
                                  isem.at[sl]).wait()

            @pl.when(g >= 2)
            def _w():
                wait_scatters(p, slpp)

            start_gathers(p, sl)

            @pl.when(g + 1 < G)
            def _pf():
                pltpu.async_copy(idx.at[blk0 + g + 1],
                                 ibuf.at[pl.ds(slpp * 2 * NB, 2 * NB)],
                                 isem.at[slpp])
            return _

        lax.fori_loop(1, G, grp, None)
        p_last = (G - 1) % 2
        drain_gather_start_scatter(p_last, (G - 1) % 3)
        wait_scatters(p_last, (G - 1) % 3)
        wait_scatters(1 - p_last, (G - 2) % 3)
        plsc.subcore_barrier()
        pltpu.sync_copy(k_acc.at[pl.ds(s * NPT, NPT)],
                        out.at[pl.ds(c * N + s * NPT, NPT)])

    return k(table, idxc, zeros)


_KB_G, _KB_NB, _KB_C = 125, 5, 40


def _kb_attn_scatter(rows3, sidx_a, sidx_b, zeros):
    @functools.partial(
        pl.kernel, mesh=_sc_mesh(),
        compiler_params=pltpu.CompilerParams(use_tc_tiling_on_sc=False),
        out_type=jax.ShapeDtypeStruct((2 * N, 16), F32),
        scratch_types=[
            pltpu.VMEM((_KB_NB, _KB_C), jnp.int32),
            pltpu.VMEM((_KB_NB, _KB_C), jnp.int32),
            pltpu.VMEM((_KB_NB, _KB_C, 16), F32),
            pltpu.VMEM_SHARED((N, 16), F32),
            pltpu.SemaphoreType.DMA((_KB_NB,)),
            pltpu.SemaphoreType.DMA((_KB_NB,)),
            pltpu.SemaphoreType.DMA((_KB_NB,)),
        ],
    )
    def k(rws, sia, sib, z, out, abuf, bbuf, rows, acc, lsem, asem, bsem):
        c = lax.axis_index("c")
        s = lax.axis_index("s")
        pltpu.sync_copy(z.at[pl.ds(s * NPT, NPT)], acc.at[pl.ds(s * NPT, NPT)])
        plsc.subcore_barrier()

        def grp(g, _):
            base = c * 10000 + s * 625 + g * _KB_NB
            pltpu.sync_copy(sia.at[pl.ds(base, _KB_NB)], abuf)
            pltpu.sync_copy(sib.at[pl.ds(base, _KB_NB)], bbuf)
            pltpu.async_copy(rws.at[pl.ds(base, _KB_NB)], rows, lsem.at[0])
            pltpu.make_async_copy(rws.at[pl.ds(base, _KB_NB)], rows,
                                  lsem.at[0]).wait()
            for b in range(_KB_NB):
                pltpu.async_copy(rows.at[b], acc.at[abuf.at[b]], asem.at[b],
                                 add=True)
            for b in range(_KB_NB):
                pltpu.make_async_copy(rows.at[b], acc.at[abuf.at[b]],
                                      asem.at[b]).wait()
                pltpu.async_copy(rows.at[b], acc.at[bbuf.at[b]], bsem.at[b],
                                 add=True)
            for b in range(_KB_NB):
                pltpu.make_async_copy(rows.at[b], acc.at[bbuf.at[b]],
                                      bsem.at[b]).wait()
            return _

        lax.fori_loop(0, _KB_G, grp, None)
        plsc.subcore_barrier()
        pltpu.sync_copy(acc.at[pl.ds(s * NPT, NPT)],
                        out.at[pl.ds(c * N + s * NPT, NPT)])

    return k(rows3, sidx_a, sidx_b, zeros)


def _tc_nodeinfo(status, est_size):
    g = N // BN

    def body(st_ref, es_ref, ni_ref, ep_ref):
        st = st_ref[0, 0]
        es = es_ref[0, 0]
        one = jnp.ones((BN, 1), F32)
        cols = [one]
        for v in (-1, 0, 1):
            cols.append((st == v).astype(F32)[:, None])
        cols.append(jnp.zeros((BN, 4), F32))
        ni_ref[...] = jnp.concatenate(cols, axis=1)
        p = jnp.concatenate(
            [jnp.sum(es)[None], jnp.sum(es * es)[None], jnp.zeros((6,), F32)])
        ep_ref[...] = p.reshape(1, 1, 8)

    return pl.pallas_call(
        body,
        grid=(g,),
        compiler_params=pltpu.CompilerParams(
            vmem_limit_bytes=60 * 1024 * 1024),
        in_specs=[
            pl.BlockSpec((1, 1, BN), lambda i: (i, 0, 0)),
            pl.BlockSpec((1, 1, BN), lambda i: (i, 0, 0)),
        ],
        out_specs=[
            pl.BlockSpec((BN, 8), lambda i: (i, 0)),
            pl.BlockSpec((1, 1, 8), lambda i: (i, 0, 0)),
        ],
        out_shape=[
            jax.ShapeDtypeStruct((N, 8), F32),
            jax.ShapeDtypeStruct((g, 1, 8), F32),
        ],
    )(status, est_size)


def _tc_scores(emb1, ef2, f2, aw1, ab1, aw2, ab2):
    g = E // BE

    def body(e1_ref, f2_ref, t2_ref, w1_ref, b1_ref, w2_ref, b2_ref, o_ref):
        e1 = e1_ref[...]
        f2v = f2_ref[0, 0]
        oh = (f2v[:, None] == lax.broadcasted_iota(jnp.int32, (1, 3), 1)
              ).astype(F32)
        e2 = jnp.dot(oh, t2_ref[...], preferred_element_type=F32)
        emb8 = jnp.concatenate([e1[:, :5], e2], axis=1)
        h = jnp.tanh(jnp.dot(emb8, w1_ref[...], preferred_element_type=F32)
                     + b1_ref[...][None, :])
        sc = jnp.dot(h, w2_ref[...], preferred_element_type=F32) \
            + b2_ref[...][None, :]
        w = jnp.exp(sc)
        o_ref[...] = jnp.concatenate(
            [w, w * emb8, jnp.zeros((BE, 7), F32)], axis=1)

    return pl.pallas_call(
        body,
        grid=(g,),
        compiler_params=pltpu.CompilerParams(
            vmem_limit_bytes=60 * 1024 * 1024),
        in_specs=[
            pl.BlockSpec((BE, 8), lambda i: (i, 0)),
            pl.BlockSpec((1, 1, BE), lambda i: (i, 0, 0)),
            pl.BlockSpec((3, 3), lambda i: (0, 0)),
            pl.BlockSpec((8, 8), lambda i: (0, 0)),
            pl.BlockSpec((8,), lambda i: (0,)),
            pl.BlockSpec((8, 1), lambda i: (0, 0)),
            pl.BlockSpec((1,), lambda i: (0,)),
        ],
        out_specs=pl.BlockSpec((BE, 16), lambda i: (i, 0)),
        out_shape=jax.ShapeDtypeStruct((E, 16), F32),
    )(emb1, f2, ef2, aw1, ab1, aw2, ab2)


def _tc_feats(acc1, acc2, est, estp, conv1_w):
    g = N // BN
    gp = N // BN

    def body(a1a_ref, a1b_ref, a2a_ref, a2b_ref, es_ref, ep_ref, w_ref,
             hp_ref, dis_ref):
        ep = ep_ref[...]
        s1 = jnp.sum(ep[:, 0, 0])
        s2 = jnp.sum(ep[:, 0, 1])
        mean = s1 / N
        var = (s2 - N * mean * mean) / (N - 1)
        std = jnp.sqrt(jnp.maximum(var, 0.0))
        es = es_ref[0, 0] - mean
        es = jnp.where(std > 1e-8, es / jnp.where(std > 1e-8, std, 1.0), es)

        a1a = a1a_ref[...]
        a1b = a1b_ref[...]
        out_deg = a1a[:, 0]
        in_deg = a1b[:, 0]
        total = in_deg + out_deg
        nst = a1a[:, 1:4] + a1b[:, 1:4]
        a2 = a2a_ref[...] + a2b_ref[...]
        denom = a2[:, 0]
        vec = a2[:, 1:9]
        agg = vec / jnp.where(denom > 0, denom, 1.0)[:, None]
        feats = jnp.concatenate(
            [es[:, None], total[:, None], nst, agg], axis=1)
        h1 = jnp.dot(feats, w_ref[...], preferred_element_type=F32)
        dis = lax.rsqrt(in_deg + 1.0)
        hp = dis[:, None] * h1
        hp_ref[0] = hp[:, :32]
        hp_ref[1] = hp[:, 32:]
        dis_ref[0, 0] = dis

    return pl.pallas_call(
        body,
        grid=(g,),
        compiler_params=pltpu.CompilerParams(
            vmem_limit_bytes=60 * 1024 * 1024),
        in_specs=[
            pl.BlockSpec((BN, 8), lambda i: (i, 0)),
            pl.BlockSpec((BN, 8), lambda i: (i, 0)),
            pl.BlockSpec((BN, 16), lambda i: (i, 0)),
            pl.BlockSpec((BN, 16), lambda i: (i, 0)),
            pl.BlockSpec((1, 1, BN), lambda i: (i, 0, 0)),
            pl.BlockSpec((gp, 1, 8), lambda i: (0, 0, 0)),
            pl.BlockSpec((13, HID), lambda i: (0, 0)),
        ],
        out_specs=[
            pl.BlockSpec((2, BN, 32), lambda i: (0, i, 0)),
            pl.BlockSpec((1, 1, BN), lambda i: (i, 0, 0)),
        ],
        out_shape=[
            jax.ShapeDtypeStruct((2, N, 32), F32),
            jax.ShapeDtypeStruct((N // BN, 1, BN), F32),
        ],
    )(acc1[0], acc1[1], acc2[0], acc2[1], est, estp, conv1_w)


def _tc_layer(S, hp, dis, b_in, w_next):
    g = N // BN

    def body(sa_ref, hpa_ref, dis_ref, b_ref, w_ref, o_ref):
        sa = sa_ref[...]
        hpa = hpa_ref[...]
        dis = dis_ref[0, 0][:, None]
        pre = jnp.concatenate(
            [dis * (sa[0] + hpa[0]), dis * (sa[1] + hpa[1])], axis=1)
        x = jnp.maximum(pre + b_ref[...][None, :], 0.0)
        h = jnp.dot(x, w_ref[...], preferred_element_type=F32)
        hp = dis * h
        o_ref[0] = hp[:, :32]
        o_ref[1] = hp[:, 32:]

    return pl.pallas_call(
        body,
        grid=(g,),
        compiler_params=pltpu.CompilerParams(
            vmem_limit_bytes=60 * 1024 * 1024),
        in_specs=[
            pl.BlockSpec((2, BN, 32), lambda i: (0, i, 0)),
            pl.BlockSpec((2, BN, 32), lambda i: (0, i, 0)),
            pl.BlockSpec((1, 1, BN), lambda i: (i, 0, 0)),
            pl.BlockSpec((HID,), lambda i: (0,)),
            pl.BlockSpec((HID, HID), lambda i: (0, 0)),
        ],
        out_specs=pl.BlockSpec((2, BN, 32), lambda i: (0, i, 0)),
        out_shape=jax.ShapeDtypeStruct((2, N, 32), F32),
    )(S, hp, dis, b_in, w_next)


def _tc_x2(S, hp, dis, b_in, status):
    g = N // BN

    def body(sa_ref, hpa_ref, dis_ref, b_ref, st_ref, x_ref, p_ref):
        sa = sa_ref[...]
        hpa = hpa_ref[...]
        dis = dis_ref[0, 0][:, None]
        pre = jnp.concatenate(
            [dis * (sa[0] + hpa[0]), dis * (sa[1] + hpa[1])], axis=1)
        x = jnp.maximum(pre + b_ref[...][None, :], 0.0)
        x_ref[...] = x
        st = st_ref[0, 0]
        cnt1 = jnp.sum((st == 1).astype(F32))
        cnt0 = jnp.sum((st == 0).astype(F32))
        p = jnp.concatenate(
            [jnp.sum(x, axis=0), cnt1[None], cnt0[None],
             jnp.zeros((62,), F32)])
        p_ref[...] = p.reshape(1, 1, 128)

    return pl.pallas_call(
        body,
        grid=(g,),
        compiler_params=pltpu.CompilerParams(
            vmem_limit_bytes=60 * 1024 * 1024),
        in_specs=[
            pl.BlockSpec((2, BN, 32), lambda i: (0, i, 0)),
            pl.BlockSpec((2, BN, 32), lambda i: (0, i, 0)),
            pl.BlockSpec((1, 1, BN), lambda i: (i, 0, 0)),
            pl.BlockSpec((HID,), lambda i: (0,)),
            pl.BlockSpec((1, 1, BN), lambda i: (i, 0, 0)),
        ],
        out_specs=[
            pl.BlockSpec((BN, HID), lambda i: (i, 0)),
            pl.BlockSpec((1, 1, 128), lambda i: (i, 0, 0)),
        ],
        out_shape=[
            jax.ShapeDtypeStruct((N, HID), F32),
            jax.ShapeDtypeStruct((g, 1, 128), F32),
        ],
    )(S, hp, dis, b_in, status)


def _tc_heads(x2, partials, status, aw1, ab1, aw2, ab2):
    g = N // BN
    gp = N // BN

    def body(x_ref, p_ref, st_ref, aw1_ref, ab1_ref, aw2_ref, ab2_ref,
             lg_ref):
        p = jnp.sum(p_ref[...][:, 0, :], axis=0)
        gmean = (p[:64] / N).reshape(1, 64)
        cnt1 = p[64]
        x = x_ref[...]
        aw = aw1_ref[...]
        gterm = jnp.dot(gmean, aw[64:, :], preferred_element_type=F32) \
            + ab1_ref[...][None, :]
        h = jnp.maximum(
            jnp.dot(x, aw[:64, :], preferred_element_type=F32) + gterm, 0.0)
        raw = (jnp.dot(h, aw2_ref[...], preferred_element_type=F32)
               + ab2_ref[...][None, :])[:, 0]
        st = st_ref[0, 0]
        m1 = (st == 1).astype(F32)
        m0 = (st == 0).astype(F32)
        m = jnp.where(cnt1 > 0, m1, m0)
        lg_ref[0, 0] = raw * m + (1.0 - m) * (-1e9)

    return pl.pallas_call(
        body,
        grid=(g,),
        compiler_params=pltpu.CompilerParams(
            vmem_limit_bytes=60 * 1024 * 1024),
        in_specs=[
            pl.BlockSpec((BN, HID), lambda i: (i, 0)),
            pl.BlockSpec((gp, 1, 128), lambda i: (0, 0, 0)),
            pl.BlockSpec((1, 1, BN), lambda i: (i, 0, 0)),
            pl.BlockSpec((2 * HID, HID), lambda i: (0, 0)),
            pl.BlockSpec((HID,), lambda i: (0,)),
            pl.BlockSpec((HID, 1), lambda i: (0, 0)),
            pl.BlockSpec((1,), lambda i: (0,)),
        ],
        out_specs=pl.BlockSpec((1, 1, BN), lambda i: (i, 0, 0)),
        out_shape=jax.ShapeDtypeStruct((N // BN, 1, BN), F32),
    )(x2, partials, status, aw1, ab1, aw2, ab2)


def _mk_idxc(g2, s2):
    return jnp.concatenate(
        [g2.reshape(-1, _KA_NB, _KA_C), s2.reshape(-1, _KA_NB, _KA_C)],
        axis=1)


def kernel(status, est_size, edge_index, edge_feat1, edge_feat2, ef1_table,
           ef2_table, aw1, ab1, aw2, ab2, conv1_w, conv1_b, conv2_w, conv2_b,
           actor_w1, actor_b1, actor_w2, actor_b2,
           crit_w1, crit_b1, crit_w2, crit_b2, crit_w3, crit_b3):
    src = edge_index[0]
    dst = edge_index[1]
    st2 = status.astype(jnp.int32).reshape(N // BN, 1, BN)
    es2 = est_size.astype(F32).reshape(N // BN, 1, BN)

    nodeinfo, estp = _tc_nodeinfo(st2, es2)

    ef1p = jnp.pad(ef1_table.astype(F32), ((0, 0), (0, 3)))
    f1r = edge_feat1.astype(jnp.int32).reshape(32 * 625, _K1_C)
    emb1 = _k1_emb_gather(ef1p, f1r).reshape(E, 8)

    wemb = _tc_scores(emb1, ef2_table.astype(F32),
                      edge_feat2.astype(jnp.int32).reshape(E // BE, 1, BE),
                      aw1, ab1, aw2, ab2)

    src80 = src.astype(jnp.int32).reshape(10000, 80)
    dst80 = dst.astype(jnp.int32).reshape(10000, 80)
    z8 = jnp.zeros((N, 8), F32)
    acc1 = _ka_gather_scatter(
        nodeinfo,
        _mk_idxc(jnp.concatenate([dst80, src80], 0),
                 jnp.concatenate([src80, dst80], 0)),
        z8, 8).reshape(2, N, 8)

    src40 = src.astype(jnp.int32).reshape(20000, _KB_C)
    dst40 = dst.astype(jnp.int32).reshape(20000, _KB_C)
    z16 = jnp.zeros((N, 16), F32)
    acc2 = _kb_attn_scatter(
        wemb.reshape(20000, _KB_C, 16), src40, dst40, z16).reshape(2, N, 16)

    h1p, dis2 = _tc_feats(acc1, acc2, es2, estp, conv1_w)

    z32 = jnp.zeros((N, 32), F32)
    idxc5 = _mk_idxc(jnp.concatenate([src80, src80 + N], 0),
                     jnp.concatenate([dst80, dst80], 0))
    S1 = _ka_gather_scatter(h1p.reshape(2 * N, 32), idxc5,
                            z32, 32).reshape(2, N, 32)

    h2p = _tc_layer(S1, h1p, dis2, conv1_b, conv2_w)

    S2 = _ka_gather_scatter(h2p.reshape(2 * N, 32), idxc5,
                            z32, 32).reshape(2, N, 32)

    x2, partials = _tc_x2(S2, h2p, dis2, conv2_b, st2)

    logits = _tc_heads(x2, partials, st2,
                       actor_w1, actor_b1, actor_w2, actor_b2)

    gm = jnp.mean(x2, axis=0)
    combined = jnp.concatenate(
        [x2, jnp.broadcast_to(gm, (N, gm.shape[0]))], axis=1)
    pooled = jnp.mean(combined, axis=0)
    hv = jax.nn.relu(pooled @ crit_w1 + crit_b1)
    hv = jax.nn.relu(hv @ crit_w2 + crit_b2)
    value = (hv @ crit_w3 + crit_b3)[0]
    return (logits.reshape(N), value)


# speedup vs baseline: 27.6635x; 1.0246x over previous
import functools

import jax
import jax.numpy as jnp
from jax import lax
from jax.experimental import pallas as pl
from jax.experimental.pallas import tpu as pltpu
from jax.experimental.pallas import tpu_sc as plsc

N = 50000
E = 800000
HID = 64
MAX_ID = 10000

NC = 2
NS = 16
NPT = N // NS

BN = 5000
BE = 8000

F32 = jnp.float32


def _sc_mesh():
    return plsc.VectorSubcoreMesh(
        core_axis_name="c", subcore_axis_name="s", num_cores=NC,
        num_subcores=NS)


_K1_G, _K1_NB, _K1_C = 125, 5, 40


def _k1_emb_gather(ef1p, f1b):
    G, NB, C = _K1_G, _K1_NB, _K1_C

    @functools.partial(
        pl.kernel, mesh=_sc_mesh(),
        compiler_params=pltpu.CompilerParams(use_tc_tiling_on_sc=False),
        out_type=jax.ShapeDtypeStruct((32 * 625, C, 8), F32),
        scratch_types=[
            pltpu.VMEM((3 * NB, C), jnp.int32),
            pltpu.VMEM((2 * NB, C, 8), F32),
            pltpu.SemaphoreType.DMA((3,)),
            pltpu.SemaphoreType.DMA((2 * NB,)),
            pltpu.SemaphoreType.DMA((2 * NB,)),
        ],
    )
    def k(tab, idx, out, ibuf, rows, isem, gsem, osem):
        c = lax.axis_index("c")
        s = lax.axis_index("s")
        wid = s * NC + c
        blk0 = wid * G
        rbase = wid * 625

        def start_gathers(p, sl):
            for b in range(NB):
                pltpu.async_copy(tab.at[ibuf.at[sl * NB + b]],
                                 rows.at[p * NB + b], gsem.at[p * NB + b])

        def drain_gather_start_out(p, sl, g):
            for b in range(NB):
                pltpu.make_async_copy(tab.at[ibuf.at[sl * NB + b]],
                                      rows.at[p * NB + b],
                                      gsem.at[p * NB + b]).wait()
                pltpu.async_copy(rows.at[p * NB + b],
                                 out.at[rbase + g * NB + b],
                                 osem.at[p * NB + b])

        def wait_outs(p, g):
            for b in range(NB):
                pltpu.make_async_copy(rows.at[p * NB + b],
                                      out.at[rbase + g * NB + b],
                                      osem.at[p * NB + b]).wait()

        pltpu.sync_copy(idx.at[blk0], ibuf.at[pl.ds(0, NB)])
        start_gathers(0, 0)
        pltpu.async_copy(idx.at[blk0 + 1], ibuf.at[pl.ds(NB, NB)], isem.at[1])

        def grp(g, _):
            p = lax.rem(g, 2)
            q = 1 - p
            sl = lax.rem(g, 3)
            slp = lax.rem(g + 2, 3)
            slpp = lax.rem(g + 1, 3)
            drain_gather_start_out(q, slp, g - 1)
            pltpu.make_async_copy(idx.at[blk0 + g],
                                  ibuf.at[pl.ds(sl * NB, NB)],
                                  isem.at[sl]).wait()

            @pl.when(g >= 2)
            def _w():
                wait_outs(p, g - 2)

            start_gathers(p, sl)

            @pl.when(g + 1 < G)
            def _pf():
                pltpu.async_copy(idx.at[blk0 + g + 1],
                                 ibuf.at[pl.ds(slpp * NB, NB)],
                                 isem.at[slpp])
            return _

        lax.fori_loop(1, G, grp, None)
        p_last = (G - 1) % 2
        drain_gather_start_out(p_last, (G - 1) % 3, G - 1)
        wait_outs(p_last, G - 1)
        wait_outs(1 - p_last, G - 2)

    return k(ef1p, f1b)


_KA_G, _KA_NB, _KA_C = 125, 5, 80


def _ka_gather_scatter(table, idxc, zeros, W):
    G, NB, C = _KA_G, _KA_NB, _KA_C

    @functools.partial(
        pl.kernel, mesh=_sc_mesh(),
        compiler_params=pltpu.CompilerParams(use_tc_tiling_on_sc=False),
        out_type=jax.ShapeDtypeStruct((2 * N, W), F32),
        scratch_types=[
            pltpu.VMEM((3 * 2 * NB, C), jnp.int32),
            pltpu.VMEM((2 * NB, C, W), F32),
            pltpu.VMEM_SHARED((N, W), F32),
            pltpu.SemaphoreType.DMA((3,)),
            pltpu.SemaphoreType.DMA((2 * NB,)),
            pltpu.SemaphoreType.DMA((2 * NB,)),
        ],
    )
    def k(tab, idx, z, out, ibuf, rows, k_acc, isem, gsem, ssem):
        c = lax.axis_index("c")
        s = lax.axis_index("s")
        pltpu.sync_copy(z.at[pl.ds(s * NPT, NPT)], k_acc.at[pl.ds(s * NPT, NPT)])
        plsc.subcore_barrier()
        blk0 = (c * NS + s) * G

        def start_gathers(p, sl):
            for b in range(NB):
                pltpu.async_copy(tab.at[ibuf.at[sl * 2 * NB + b]],
                                 rows.at[p * NB + b], gsem.at[p * NB + b])

        def drain_gather_start_scatter(p, sl):
            for b in range(NB):
                pltpu.make_async_copy(tab.at[ibuf.at[sl * 2 * NB + b]],
                                      rows.at[p * NB + b],
                                      gsem.at[p * NB + b]).wait()
                pltpu.async_copy(rows.at[p * NB + b],
                                 k_acc.at[ibuf.at[sl * 2 * NB + NB + b]],
                                 ssem.at[p * NB + b], add=True)

        def wait_scatters(p, sl):
            for b in range(NB):
                pltpu.make_async_copy(rows.at[p * NB + b],
                                      k_acc.at[ibuf.at[sl * 2 * NB + NB + b]],
                                      ssem.at[p * NB + b]).wait()

        pltpu.sync_copy(idx.at[blk0], ibuf.at[pl.ds(0, 2 * NB)])
        start_gathers(0, 0)
        pltpu.async_copy(idx.at[blk0 + 1], ibuf.at[pl.ds(2 * NB, 2 * NB)],
                         isem.at[1])

        def grp(g, _):
            p = lax.rem(g, 2)
            q = 1 - p
            sl = lax.rem(g, 3)
            slp = lax.rem(g + 2, 3)
            slpp = lax.rem(g + 1, 3)
            drain_gather_start_scatter(q, slp)
            pltpu.make_async_copy(idx.at[blk0 + g],
                                  ibuf.at[pl.ds(sl * 2 * NB, 2 * NB)],
                                  isem.at[sl]).wait()

            @pl.when(g >= 2)
            def _w():
                wait_scatters(p, slpp)

            start_gathers(p, sl)

            @pl.when(g + 1 < G)
            def _pf():
                pltpu.async_copy(idx.at[blk0 + g + 1],
                                 ibuf.at[pl.ds(slpp * 2 * NB, 2 * NB)],
                                 isem.at[slpp])
            return _

        lax.fori_loop(1, G, grp, None)
        p_last = (G - 1) % 2
        drain_gather_start_scatter(p_last, (G - 1) % 3)
        wait_scatters(p_last, (G - 1) % 3)
        wait_scatters(1 - p_last, (G - 2) % 3)
        plsc.subcore_barrier()
        pltpu.sync_copy(k_acc.at[pl.ds(s * NPT, NPT)],
                        out.at[pl.ds(c * N + s * NPT, NPT)])

    return k(table, idxc, zeros)


_KB_G, _KB_NB, _KB_C = 125, 5, 40


def _kb_attn_scatter(rows3, idxc, zeros):
    G, NB, C = _KB_G, _KB_NB, _KB_C

    @functools.partial(
        pl.kernel, mesh=_sc_mesh(),
        compiler_params=pltpu.CompilerParams(use_tc_tiling_on_sc=False),
        out_type=jax.ShapeDtypeStruct((2 * N, 16), F32),
        scratch_types=[
            pltpu.VMEM((3 * 2 * NB, C), jnp.int32),
            pltpu.VMEM((2 * NB, C, 16), F32),
            pltpu.VMEM_SHARED((N, 16), F32),
            pltpu.SemaphoreType.DMA((3,)),
            pltpu.SemaphoreType.DMA((2,)),
            pltpu.SemaphoreType.DMA((2 * NB,)),
            pltpu.SemaphoreType.DMA((2 * NB,)),
        ],
    )
    def k(rws, idx, z, out, ibuf, rows, k_acc, isem, lsem, asem, bsem):
        c = lax.axis_index("c")
        s = lax.axis_index("s")
        pltpu.sync_copy(z.at[pl.ds(s * NPT, NPT)], k_acc.at[pl.ds(s * NPT, NPT)])
        plsc.subcore_barrier()
        blk0 = (c * NS + s) * G
        rbase = (c * NS + s) * G * NB

        def start_load(p, g):
            pltpu.async_copy(rws.at[pl.ds(rbase + g * NB, NB)],
                             rows.at[pl.ds(p * NB, NB)], lsem.at[p])

        def drain_load_start_scatters(p, sl, g):
            pltpu.make_async_copy(rws.at[pl.ds(rbase + g * NB, NB)],
                                  rows.at[pl.ds(p * NB, NB)],
                                  lsem.at[p]).wait()
            for b in range(NB):
                pltpu.async_copy(rows.at[p * NB + b],
                                 k_acc.at[ibuf.at[sl * 2 * NB + b]],
                                 asem.at[p * NB + b], add=True)
                pltpu.async_copy(rows.at[p * NB + b],
                                 k_acc.at[ibuf.at[sl * 2 * NB + NB + b]],
                                 bsem.at[p * NB + b], add=True)

        def wait_scatters(p, sl):
            for b in range(NB):
                pltpu.make_async_copy(rows.at[p * NB + b],
                                      k_acc.at[ibuf.at[sl * 2 * NB + b]],
                                      asem.at[p * NB + b]).wait()
                pltpu.make_async_copy(rows.at[p * NB + b],
                                      k_acc.at[ibuf.at[sl * 2 * NB + NB + b]],
                                      bsem.at[p * NB + b]).wait()

        pltpu.sync_copy(idx.at[blk0], ibuf.at[pl.ds(0, 2 * NB)])
        start_load(0, 0)
        pltpu.async_copy(idx.at[blk0 + 1], ibuf.at[pl.ds(2 * NB, 2 * NB)],
                         isem.at[1])

        def grp(g, _):
            p = lax.rem(g, 2)
            q = 1 - p
            sl = lax.rem(g, 3)
            slp = lax.rem(g + 2, 3)
            slpp = lax.rem(g + 1, 3)
            drain_load_start_scatters(q, slp, g - 1)
            pltpu.make_async_copy(idx.at[blk0 + g],
                                  ibuf.at[pl.ds(sl * 2 * NB, 2 * NB)],
                                  isem.at[sl]).wait()

            @pl.when(g >= 2)
            def _w():
                wait_scatters(p, slpp)

            start_load(p, g)

            @pl.when(g + 1 < G)
            def _pf():
                pltpu.async_copy(idx.at[blk0 + g + 1],
                                 ibuf.at[pl.ds(slpp * 2 * NB, 2 * NB)],
                                 isem.at[slpp])
            return _

        lax.fori_loop(1, G, grp, None)
        p_last = (G - 1) % 2
        drain_load_start_scatters(p_last, (G - 1) % 3, G - 1)
        wait_scatters(p_last, (G - 1) % 3)
        wait_scatters(1 - p_last, (G - 2) % 3)
        plsc.subcore_barrier()
        pltpu.sync_copy(k_acc.at[pl.ds(s * NPT, NPT)],
                        out.at[pl.ds(c * N + s * NPT, NPT)])

    return k(rows3, idxc, zeros)


def _tc_nodeinfo(status, est_size):
    g = N // BN

    def body(st_ref, es_ref, ni_ref, ep_ref):
        st = st_ref[0, 0]
        es = es_ref[0, 0]
        one = jnp.ones((BN, 1), F32)
        cols = [one]
        for v in (-1, 0, 1):
            cols.append((st == v).astype(F32)[:, None])
        cols.append(jnp.zeros((BN, 4), F32))
        ni_ref[...] = jnp.concatenate(cols, axis=1)
        p = jnp.concatenate(
            [jnp.sum(es)[None], jnp.sum(es * es)[None], jnp.zeros((6,), F32)])
        ep_ref[...] = p.reshape(1, 1, 8)

    return pl.pallas_call(
        body,
        grid=(g,),
        compiler_params=pltpu.CompilerParams(
            vmem_limit_bytes=60 * 1024 * 1024),
        in_specs=[
            pl.BlockSpec((1, 1, BN), lambda i: (i, 0, 0)),
            pl.BlockSpec((1, 1, BN), lambda i: (i, 0, 0)),
        ],
        out_specs=[
            pl.BlockSpec((BN, 8), lambda i: (i, 0)),
            pl.BlockSpec((1, 1, 8), lambda i: (i, 0, 0)),
        ],
        out_shape=[
            jax.ShapeDtypeStruct((N, 8), F32),
            jax.ShapeDtypeStruct((g, 1, 8), F32),
        ],
    )(status, est_size)


def _tc_scores(emb1, ef2, f2, aw1, ab1, aw2, ab2):
    g = E // BE

    def body(e1_ref, f2_ref, t2_ref, w1_ref, b1_ref, w2_ref, b2_ref, o_ref):
        e1 = e1_ref[...]
        f2v = f2_ref[0, 0]
        oh = (f2v[:, None] == lax.broadcasted_iota(jnp.int32, (1, 3), 1)
              ).astype(F32)
        e2 = jnp.dot(oh, t2_ref[...], preferred_element_type=F32)
        emb8 = jnp.concatenate([e1[:, :5], e2], axis=1)
        h = jnp.tanh(jnp.dot(emb8, w1_ref[...], preferred_element_type=F32)
                     + b1_ref[...][None, :])
        sc = jnp.dot(h, w2_ref[...], preferred_element_type=F32) \
            + b2_ref[...][None, :]
        w = jnp.exp(sc)
        o_ref[...] = jnp.concatenate(
            [w, w * emb8, jnp.zeros((BE, 7), F32)], axis=1)

    return pl.pallas_call(
        body,
        grid=(g,),
        compiler_params=pltpu.CompilerParams(
            vmem_limit_bytes=60 * 1024 * 1024),
        in_specs=[
            pl.BlockSpec((BE, 8), lambda i: (i, 0)),
            pl.BlockSpec((1, 1, BE), lambda i: (i, 0, 0)),
            pl.BlockSpec((3, 3), lambda i: (0, 0)),
            pl.BlockSpec((8, 8), lambda i: (0, 0)),
            pl.BlockSpec((8,), lambda i: (0,)),
            pl.BlockSpec((8, 1), lambda i: (0, 0)),
            pl.BlockSpec((1,), lambda i: (0,)),
        ],
        out_specs=pl.BlockSpec((BE, 16), lambda i: (i, 0)),
        out_shape=jax.ShapeDtypeStruct((E, 16), F32),
    )(emb1, f2, ef2, aw1, ab1, aw2, ab2)


def _tc_feats(acc1, acc2, est, estp, conv1_w):
    g = N // BN
    gp = N // BN

    def body(a1a_ref, a1b_ref, a2a_ref, a2b_ref, es_ref, ep_ref, w_ref,
             hp_ref, dis_ref):
        ep = ep_ref[...]
        s1 = jnp.sum(ep[:, 0, 0])
        s2 = jnp.sum(ep[:, 0, 1])
        mean = s1 / N
        var = (s2 - N * mean * mean) / (N - 1)
        std = jnp.sqrt(jnp.maximum(var, 0.0))
        es = es_ref[0, 0] - mean
        es = jnp.where(std > 1e-8, es / jnp.where(std > 1e-8, std, 1.0), es)

        a1a = a1a_ref[...]
        a1b = a1b_ref[...]
        out_deg = a1a[:, 0]
        in_deg = a1b[:, 0]
        total = in_deg + out_deg
        nst = a1a[:, 1:4] + a1b[:, 1:4]
        a2 = a2a_ref[...] + a2b_ref[...]
        denom = a2[:, 0]
        vec = a2[:, 1:9]
        agg = vec / jnp.where(denom > 0, denom, 1.0)[:, None]
        feats = jnp.concatenate(
            [es[:, None], total[:, None], nst, agg], axis=1)
        h1 = jnp.dot(feats, w_ref[...], preferred_element_type=F32)
        dis = lax.rsqrt(in_deg + 1.0)
        hp = dis[:, None] * h1
        hp_ref[0] = hp[:, :32]
        hp_ref[1] = hp[:, 32:]
        dis_ref[0, 0] = dis

    return pl.pallas_call(
        body,
        grid=(g,),
        compiler_params=pltpu.CompilerParams(
            vmem_limit_bytes=60 * 1024 * 1024),
        in_specs=[
            pl.BlockSpec((BN, 8), lambda i: (i, 0)),
            pl.BlockSpec((BN, 8), lambda i: (i, 0)),
            pl.BlockSpec((BN, 16), lambda i: (i, 0)),
            pl.BlockSpec((BN, 16), lambda i: (i, 0)),
            pl.BlockSpec((1, 1, BN), lambda i: (i, 0, 0)),
            pl.BlockSpec((gp, 1, 8), lambda i: (0, 0, 0)),
            pl.BlockSpec((13, HID), lambda i: (0, 0)),
        ],
        out_specs=[
            pl.BlockSpec((2, BN, 32), lambda i: (0, i, 0)),
            pl.BlockSpec((1, 1, BN), lambda i: (i, 0, 0)),
        ],
        out_shape=[
            jax.ShapeDtypeStruct((2, N, 32), F32),
            jax.ShapeDtypeStruct((N // BN, 1, BN), F32),
        ],
    )(acc1[0], acc1[1], acc2[0], acc2[1], est, estp, conv1_w)


def _tc_layer(S, hp, dis, b_in, w_next):
    g = N // BN

    def body(sa_ref, hpa_ref, dis_ref, b_ref, w_ref, o_ref):
        sa = sa_ref[...]
        hpa = hpa_ref[...]
        dis = dis_ref[0, 0][:, None]
        pre = jnp.concatenate(
            [dis * (sa[0] + hpa[0]), dis * (sa[1] + hpa[1])], axis=1)
        x = jnp.maximum(pre + b_ref[...][None, :], 0.0)
        h = jnp.dot(x, w_ref[...], preferred_element_type=F32)
        hp = dis * h
        o_ref[0] = hp[:, :32]
        o_ref[1] = hp[:, 32:]

    return pl.pallas_call(
        body,
        grid=(g,),
        compiler_params=pltpu.CompilerParams(
            vmem_limit_bytes=60 * 1024 * 1024),
        in_specs=[
            pl.BlockSpec((2, BN, 32), lambda i: (0, i, 0)),
            pl.BlockSpec((2, BN, 32), lambda i: (0, i, 0)),
            pl.BlockSpec((1, 1, BN), lambda i: (i, 0, 0)),
            pl.BlockSpec((HID,), lambda i: (0,)),
            pl.BlockSpec((HID, HID), lambda i: (0, 0)),
        ],
        out_specs=pl.BlockSpec((2, BN, 32), lambda i: (0, i, 0)),
        out_shape=jax.ShapeDtypeStruct((2, N, 32), F32),
    )(S, hp, dis, b_in, w_next)


def _tc_x2(S, hp, dis, b_in, status):
    g = N // BN

    def body(sa_ref, hpa_ref, dis_ref, b_ref, st_ref, x_ref, p_ref):
        sa = sa_ref[...]
        hpa = hpa_ref[...]
        dis = dis_ref[0, 0][:, None]
        pre = jnp.concatenate(
            [dis * (sa[0] + hpa[0]), dis * (sa[1] + hpa[1])], axis=1)
        x = jnp.maximum(pre + b_ref[...][None, :], 0.0)
        x_ref[...] = x
        st = st_ref[0, 0]
        cnt1 = jnp.sum((st == 1).astype(F32))
        cnt0 = jnp.sum((st == 0).astype(F32))
        p = jnp.concatenate(
            [jnp.sum(x, axis=0), cnt1[None], cnt0[None],
             jnp.zeros((62,), F32)])
        p_ref[...] = p.reshape(1, 1, 128)

    return pl.pallas_call(
        body,
        grid=(g,),
        compiler_params=pltpu.CompilerParams(
            vmem_limit_bytes=60 * 1024 * 1024),
        in_specs=[
            pl.BlockSpec((2, BN, 32), lambda i: (0, i, 0)),
            pl.BlockSpec((2, BN, 32), lambda i: (0, i, 0)),
            pl.BlockSpec((1, 1, BN), lambda i: (i, 0, 0)),
            pl.BlockSpec((HID,), lambda i: (0,)),
            pl.BlockSpec((1, 1, BN), lambda i: (i, 0, 0)),
        ],
        out_specs=[
            pl.BlockSpec((BN, HID), lambda i: (i, 0)),
            pl.BlockSpec((1, 1, 128), lambda i: (i, 0, 0)),
        ],
        out_shape=[
            jax.ShapeDtypeStruct((N, HID), F32),
            jax.ShapeDtypeStruct((g, 1, 128), F32),
        ],
    )(S, hp, dis, b_in, status)


def _tc_heads(x2, partials, status, aw1, ab1, aw2, ab2):
    g = N // BN
    gp = N // BN

    def body(x_ref, p_ref, st_ref, aw1_ref, ab1_ref, aw2_ref, ab2_ref,
             lg_ref):
        p = jnp.sum(p_ref[...][:, 0, :], axis=0)
        gmean = (p[:64] / N).reshape(1, 64)
        cnt1 = p[64]
        x = x_ref[...]
        aw = aw1_ref[...]
        gterm = jnp.dot(gmean, aw[64:, :], preferred_element_type=F32) \
            + ab1_ref[...][None, :]
        h = jnp.maximum(
            jnp.dot(x, aw[:64, :], preferred_element_type=F32) + gterm, 0.0)
        raw = (jnp.dot(h, aw2_ref[...], preferred_element_type=F32)
               + ab2_ref[...][None, :])[:, 0]
        st = st_ref[0, 0]
        m1 = (st == 1).astype(F32)
        m0 = (st == 0).astype(F32)
        m = jnp.where(cnt1 > 0, m1, m0)
        lg_ref[0, 0] = raw * m + (1.0 - m) * (-1e9)

    return pl.pallas_call(
        body,
        grid=(g,),
        compiler_params=pltpu.CompilerParams(
            vmem_limit_bytes=60 * 1024 * 1024),
        in_specs=[
            pl.BlockSpec((BN, HID), lambda i: (i, 0)),
            pl.BlockSpec((gp, 1, 128), lambda i: (0, 0, 0)),
            pl.BlockSpec((1, 1, BN), lambda i: (i, 0, 0)),
            pl.BlockSpec((2 * HID, HID), lambda i: (0, 0)),
            pl.BlockSpec((HID,), lambda i: (0,)),
            pl.BlockSpec((HID, 1), lambda i: (0, 0)),
            pl.BlockSpec((1,), lambda i: (0,)),
        ],
        out_specs=pl.BlockSpec((1, 1, BN), lambda i: (i, 0, 0)),
        out_shape=jax.ShapeDtypeStruct((N // BN, 1, BN), F32),
    )(x2, partials, status, aw1, ab1, aw2, ab2)


def _mk_idxc(g2, s2):
    return jnp.concatenate(
        [g2.reshape(-1, _KA_NB, _KA_C), s2.reshape(-1, _KA_NB, _KA_C)],
        axis=1)


def kernel(status, est_size, edge_index, edge_feat1, edge_feat2, ef1_table,
           ef2_table, aw1, ab1, aw2, ab2, conv1_w, conv1_b, conv2_w, conv2_b,
           actor_w1, actor_b1, actor_w2, actor_b2,
           crit_w1, crit_b1, crit_w2, crit_b2, crit_w3, crit_b3):
    src = edge_index[0]
    dst = edge_index[1]
    st2 = status.astype(jnp.int32).reshape(N // BN, 1, BN)
    es2 = est_size.astype(F32).reshape(N // BN, 1, BN)

    nodeinfo, estp = _tc_nodeinfo(st2, es2)

    ef1p = jnp.pad(ef1_table.astype(F32), ((0, 0), (0, 3)))
    f1b = edge_feat1.astype(jnp.int32).reshape(32 * _K1_G, _K1_NB, _K1_C)
    emb1 = _k1_emb_gather(ef1p, f1b).reshape(E, 8)

    wemb = _tc_scores(emb1, ef2_table.astype(F32),
                      edge_feat2.astype(jnp.int32).reshape(E // BE, 1, BE),
                      aw1, ab1, aw2, ab2)

    src80 = src.astype(jnp.int32).reshape(10000, 80)
    dst80 = dst.astype(jnp.int32).reshape(10000, 80)
    z8 = jnp.zeros((N, 8), F32)
    acc1 = _ka_gather_scatter(
        nodeinfo,
        _mk_idxc(jnp.concatenate([dst80, src80], 0),
                 jnp.concatenate([src80, dst80], 0)),
        z8, 8).reshape(2, N, 8)

    src40 = src.astype(jnp.int32).reshape(-1, _KB_NB, _KB_C)
    dst40 = dst.astype(jnp.int32).reshape(-1, _KB_NB, _KB_C)
    z16 = jnp.zeros((N, 16), F32)
    acc2 = _kb_attn_scatter(
        wemb.reshape(20000, _KB_C, 16),
        jnp.concatenate([src40, dst40], axis=1), z16).reshape(2, N, 16)

    h1p, dis2 = _tc_feats(acc1, acc2, es2, estp, conv1_w)

    z32 = jnp.zeros((N, 32), F32)
    idxc5 = _mk_idxc(jnp.concatenate([src80, src80 + N], 0),
                     jnp.concatenate([dst80, dst80], 0))
    S1 = _ka_gather_scatter(h1p.reshape(2 * N, 32), idxc5,
                            z32, 32).reshape(2, N, 32)

    h2p = _tc_layer(S1, h1p, dis2, conv1_b, conv2_w)

    S2 = _ka_gather_scatter(h2p.reshape(2 * N, 32), idxc5,
                            z32, 32).reshape(2, N, 32)

    x2, partials = _tc_x2(S2, h2p, dis2, conv2_b, st2)

    logits = _tc_heads(x2, partials, st2,
                       actor_w1, actor_b1, actor_w2, actor_b2)

    gm = jnp.mean(x2, axis=0)
    combined = jnp.concatenate(
        [x2, jnp.broadcast_to(gm, (N, gm.shape[0]))], axis=1)
    pooled = jnp.mean(combined, axis=0)
    hv = jax.nn.relu(pooled @ crit_w1 + crit_b1)
    hv = jax.nn.relu(hv @ crit_w2 + crit_b2)
    value = (hv @ crit_w3 + crit_b3)[0]
    return (logits.reshape(N), value)


# speedup vs baseline: 27.7535x; 1.0033x over previous
import functools

import jax
import jax.numpy as jnp
from jax import lax
from jax.experimental import pallas as pl
from jax.experimental.pallas import tpu as pltpu
from jax.experimental.pallas import tpu_sc as plsc

N = 50000
E = 800000
HID = 64
MAX_ID = 10000

NC = 2
NS = 16
NPT = N // NS

BN = 5000
BE = 16000

F32 = jnp.float32


def _sc_mesh():
    return plsc.VectorSubcoreMesh(
        core_axis_name="c", subcore_axis_name="s", num_cores=NC,
        num_subcores=NS)


_K1_G, _K1_NB, _K1_C = 125, 5, 40


def _k1_emb_gather(ef1p, f1b):
    G, NB, C = _K1_G, _K1_NB, _K1_C

    @functools.partial(
        pl.kernel, mesh=_sc_mesh(),
        compiler_params=pltpu.CompilerParams(use_tc_tiling_on_sc=False),
        out_type=jax.ShapeDtypeStruct((32 * 625, C, 8), F32),
        scratch_types=[
            pltpu.VMEM((3 * NB, C), jnp.int32),
            pltpu.VMEM((2 * NB, C, 8), F32),
            pltpu.SemaphoreType.DMA((3,)),
            pltpu.SemaphoreType.DMA((2 * NB,)),
            pltpu.SemaphoreType.DMA((2 * NB,)),
        ],
    )
    def k(tab, idx, out, ibuf, rows, isem, gsem, osem):
        c = lax.axis_index("c")
        s = lax.axis_index("s")
        wid = s * NC + c
        blk0 = wid * G
        rbase = wid * 625

        def start_gathers(p, sl):
            for b in range(NB):
                pltpu.async_copy(tab.at[ibuf.at[sl * NB + b]],
                                 rows.at[p * NB + b], gsem.at[p * NB + b])

        def drain_gather_start_out(p, sl, g):
            for b in range(NB):
                pltpu.make_async_copy(tab.at[ibuf.at[sl * NB + b]],
                                      rows.at[p * NB + b],
                                      gsem.at[p * NB + b]).wait()
                pltpu.async_copy(rows.at[p * NB + b],
                                 out.at[rbase + g * NB + b],
                                 osem.at[p * NB + b])

        def wait_outs(p, g):
            for b in range(NB):
                pltpu.make_async_copy(rows.at[p * NB + b],
                                      out.at[rbase + g * NB + b],
                                      osem.at[p * NB + b]).wait()

        pltpu.sync_copy(idx.at[blk0], ibuf.at[pl.ds(0, NB)])
        start_gathers(0, 0)
        pltpu.async_copy(idx.at[blk0 + 1], ibuf.at[pl.ds(NB, NB)], isem.at[1])

        def grp(g, _):
            p = lax.rem(g, 2)
            q = 1 - p
            sl = lax.rem(g, 3)
            slp = lax.rem(g + 2, 3)
            slpp = lax.rem(g + 1, 3)
            drain_gather_start_out(q, slp, g - 1)
            pltpu.make_async_copy(idx.at[blk0 + g],
                                  ibuf.at[pl.ds(sl * NB, NB)],
                                  isem.at[sl]).wait()

            @pl.when(g >= 2)
            def _w():
                wait_outs(p, g - 2)

            start_gathers(p, sl)

            @pl.when(g + 1 < G)
            def _pf():
                pltpu.async_copy(idx.at[blk0 + g + 1],
                                 ibuf.at[pl.ds(slpp * NB, NB)],
                                 isem.at[slpp])
            return _

        lax.fori_loop(1, G, grp, None)
        p_last = (G - 1) % 2
        drain_gather_start_out(p_last, (G - 1) % 3, G - 1)
        wait_outs(p_last, G - 1)
        wait_outs(1 - p_last, G - 2)

    return k(ef1p, f1b)


_KA_G, _KA_NB, _KA_C = 125, 5, 80


def _ka_gather_scatter(table, idxc, zeros, W):
    G, NB, C = _KA_G, _KA_NB, _KA_C

    @functools.partial(
        pl.kernel, mesh=_sc_mesh(),
        compiler_params=pltpu.CompilerParams(use_tc_tiling_on_sc=False),
        out_type=jax.ShapeDtypeStruct((2 * N, W), F32),
        scratch_types=[
            pltpu.VMEM((3 * 2 * NB, C), jnp.int32),
            pltpu.VMEM((2 * NB, C, W), F32),
            pltpu.VMEM_SHARED((N, W), F32),
            pltpu.SemaphoreType.DMA((3,)),
            pltpu.SemaphoreType.DMA((2 * NB,)),
            pltpu.SemaphoreType.DMA((2 * NB,)),
        ],
    )
    def k(tab, idx, z, out, ibuf, rows, k_acc, isem, gsem, ssem):
        c = lax.axis_index("c")
        s = lax.axis_index("s")
        pltpu.sync_copy(z.at[pl.ds(s * NPT, NPT)], k_acc.at[pl.ds(s * NPT, NPT)])
        plsc.subcore_barrier()
        blk0 = (c * NS + s) * G

        def start_gathers(p, sl):
            for b in range(NB):
                pltpu.async_copy(tab.at[ibuf.at[sl * 2 * NB + b]],
                                 rows.at[p * NB + b], gsem.at[p * NB + b])

        def drain_gather_start_scatter(p, sl):
            for b in range(NB):
                pltpu.make_async_copy(tab.at[ibuf.at[sl * 2 * NB + b]],
                                      rows.at[p * NB + b],
                                      gsem.at[p * NB + b]).wait()
                pltpu.async_copy(rows.at[p * NB + b],
                                 k_acc.at[ibuf.at[sl * 2 * NB + NB + b]],
                                 ssem.at[p * NB + b], add=True)

        def wait_scatters(p, sl):
            for b in range(NB):
                pltpu.make_async_copy(rows.at[p * NB + b],
                                      k_acc.at[ibuf.at[sl * 2 * NB + NB + b]],
                                      ssem.at[p * NB + b]).wait()

        pltpu.sync_copy(idx.at[blk0], ibuf.at[pl.ds(0, 2 * NB)])
        start_gathers(0, 0)
        pltpu.async_copy(idx.at[blk0 + 1], ibuf.at[pl.ds(2 * NB, 2 * NB)],
                         isem.at[1])

        def grp(g, _):
            p = lax.rem(g, 2)
            q = 1 - p
            sl = lax.rem(g, 3)
            slp = lax.rem(g + 2, 3)
            slpp = lax.rem(g + 1, 3)
            drain_gather_start_scatter(q, slp)
            pltpu.make_async_copy(idx.at[blk0 + g],
                                  ibuf.at[pl.ds(sl * 2 * NB, 2 * NB)],
                                  isem.at[sl]).wait()

            @pl.when(g >= 2)
            def _w():
                wait_scatters(p, slpp)

            start_gathers(p, sl)

            @pl.when(g + 1 < G)
            def _pf():
                pltpu.async_copy(idx.at[blk0 + g + 1],
                                 ibuf.at[pl.ds(slpp * 2 * NB, 2 * NB)],
                                 isem.at[slpp])
            return _

        lax.fori_loop(1, G, grp, None)
        p_last = (G - 1) % 2
        drain_gather_start_scatter(p_last, (G - 1) % 3)
        wait_scatters(p_last, (G - 1) % 3)
        wait_scatters(1 - p_last, (G - 2) % 3)
        plsc.subcore_barrier()
        pltpu.sync_copy(k_acc.at[pl.ds(s * NPT, NPT)],
                        out.at[pl.ds(c * N + s * NPT, NPT)])

    return k(table, idxc, zeros)


_KB_G, _KB_NB, _KB_C = 125, 5, 40


def _kb_attn_scatter(rows3, idxc, zeros):
    G, NB, C = _KB_G, _KB_NB, _KB_C

    @functools.partial(
        pl.kernel, mesh=_sc_mesh(),
        compiler_params=pltpu.CompilerParams(use_tc_tiling_on_sc=False),
        out_type=jax.ShapeDtypeStruct((2 * N, 16), F32),
        scratch_types=[
            pltpu.VMEM((3 * 2 * NB, C), jnp.int32),
            pltpu.VMEM((2 * NB, C, 16), F32),
            pltpu.VMEM_SHARED((N, 16), F32),
            pltpu.SemaphoreType.DMA((3,)),
            pltpu.SemaphoreType.DMA((2,)),
            pltpu.SemaphoreType.DMA((2 * NB,)),
            pltpu.SemaphoreType.DMA((2 * NB,)),
        ],
    )
    def k(rws, idx, z, out, ibuf, rows, k_acc, isem, lsem, asem, bsem):
        c = lax.axis_index("c")
        s = lax.axis_index("s")
        pltpu.sync_copy(z.at[pl.ds(s * NPT, NPT)], k_acc.at[pl.ds(s * NPT, NPT)])
        plsc.subcore_barrier()
        blk0 = (c * NS + s) * G
        rbase = (c * NS + s) * G * NB

        def start_load(p, g):
            pltpu.async_copy(rws.at[pl.ds(rbase + g * NB, NB)],
                             rows.at[pl.ds(p * NB, NB)], lsem.at[p])

        def drain_load_start_scatters(p, sl, g):
            pltpu.make_async_copy(rws.at[pl.ds(rbase + g * NB, NB)],
                                  rows.at[pl.ds(p * NB, NB)],
                                  lsem.at[p]).wait()
            for b in range(NB):
                pltpu.async_copy(rows.at[p * NB + b],
                                 k_acc.at[ibuf.at[sl * 2 * NB + b]],
                                 asem.at[p * NB + b], add=True)
                pltpu.async_copy(rows.at[p * NB + b],
                                 k_acc.at[ibuf.at[sl * 2 * NB + NB + b]],
                                 bsem.at[p * NB + b], add=True)

        def wait_scatters(p, sl):
            for b in range(NB):
                pltpu.make_async_copy(rows.at[p * NB + b],
                                      k_acc.at[ibuf.at[sl * 2 * NB + b]],
                                      asem.at[p * NB + b]).wait()
                pltpu.make_async_copy(rows.at[p * NB + b],
                                      k_acc.at[ibuf.at[sl * 2 * NB + NB + b]],
                                      bsem.at[p * NB + b]).wait()

        pltpu.sync_copy(idx.at[blk0], ibuf.at[pl.ds(0, 2 * NB)])
        start_load(0, 0)
        pltpu.async_copy(idx.at[blk0 + 1], ibuf.at[pl.ds(2 * NB, 2 * NB)],
                         isem.at[1])

        def grp(g, _):
            p = lax.rem(g, 2)
            q = 1 - p
            sl = lax.rem(g, 3)
            slp = lax.rem(g + 2, 3)
            slpp = lax.rem(g + 1, 3)
            drain_load_start_scatters(q, slp, g - 1)
            pltpu.make_async_copy(idx.at[blk0 + g],
                                  ibuf.at[pl.ds(sl * 2 * NB, 2 * NB)],
                                  isem.at[sl]).wait()

            @pl.when(g >= 2)
            def _w():
                wait_scatters(p, slpp)

            start_load(p, g)

            @pl.when(g + 1 < G)
            def _pf():
                pltpu.async_copy(idx.at[blk0 + g + 1],
                                 ibuf.at[pl.ds(slpp * 2 * NB, 2 * NB)],
                                 isem.at[slpp])
            return _

        lax.fori_loop(1, G, grp, None)
        p_last = (G - 1) % 2
        drain_load_start_scatters(p_last, (G - 1) % 3, G - 1)
        wait_scatters(p_last, (G - 1) % 3)
        wait_scatters(1 - p_last, (G - 2) % 3)
        plsc.subcore_barrier()
        pltpu.sync_copy(k_acc.at[pl.ds(s * NPT, NPT)],
                        out.at[pl.ds(c * N + s * NPT, NPT)])

    return k(rows3, idxc, zeros)


def _tc_nodeinfo(status, est_size):
    g = N // BN

    def body(st_ref, es_ref, ni_ref, ep_ref):
        st = st_ref[0, 0]
        es = es_ref[0, 0]
        one = jnp.ones((BN, 1), F32)
        cols = [one]
        for v in (-1, 0, 1):
            cols.append((st == v).astype(F32)[:, None])
        cols.append(jnp.zeros((BN, 4), F32))
        ni_ref[...] = jnp.concatenate(cols, axis=1)
        p = jnp.concatenate(
            [jnp.sum(es)[None], jnp.sum(es * es)[None], jnp.zeros((6,), F32)])
        ep_ref[...] = p.reshape(1, 1, 8)

    return pl.pallas_call(
        body,
        grid=(g,),
        compiler_params=pltpu.CompilerParams(
            vmem_limit_bytes=60 * 1024 * 1024),
        in_specs=[
            pl.BlockSpec((1, 1, BN), lambda i: (i, 0, 0)),
            pl.BlockSpec((1, 1, BN), lambda i: (i, 0, 0)),
        ],
        out_specs=[
            pl.BlockSpec((BN, 8), lambda i: (i, 0)),
            pl.BlockSpec((1, 1, 8), lambda i: (i, 0, 0)),
        ],
        out_shape=[
            jax.ShapeDtypeStruct((N, 8), F32),
            jax.ShapeDtypeStruct((g, 1, 8), F32),
        ],
    )(status, est_size)


def _tc_scores(emb1, ef2, f2, aw1, ab1, aw2, ab2):
    g = E // BE

    def body(e1_ref, f2_ref, t2_ref, w1_ref, b1_ref, w2_ref, b2_ref, o_ref):
        e1 = e1_ref[...]
        f2v = f2_ref[0, 0]
        oh = (f2v[:, None] == lax.broadcasted_iota(jnp.int32, (1, 3), 1)
              ).astype(F32)
        e2 = jnp.dot(oh, t2_ref[...], preferred_element_type=F32)
        emb8 = jnp.concatenate([e1[:, :5], e2], axis=1)
        h = jnp.tanh(jnp.dot(emb8, w1_ref[...], preferred_element_type=F32)
                     + b1_ref[...][None, :])
        sc = jnp.dot(h, w2_ref[...], preferred_element_type=F32) \
            + b2_ref[...][None, :]
        w = jnp.exp(sc)
        o_ref[...] = jnp.concatenate(
            [w, w * emb8, jnp.zeros((BE, 7), F32)], axis=1)

    return pl.pallas_call(
        body,
        grid=(g,),
        compiler_params=pltpu.CompilerParams(
            vmem_limit_bytes=60 * 1024 * 1024),
        in_specs=[
            pl.BlockSpec((BE, 8), lambda i: (i, 0)),
            pl.BlockSpec((1, 1, BE), lambda i: (i, 0, 0)),
            pl.BlockSpec((3, 3), lambda i: (0, 0)),
            pl.BlockSpec((8, 8), lambda i: (0, 0)),
            pl.BlockSpec((8,), lambda i: (0,)),
            pl.BlockSpec((8, 1), lambda i: (0, 0)),
            pl.BlockSpec((1,), lambda i: (0,)),
        ],
        out_specs=pl.BlockSpec((BE, 16), lambda i: (i, 0)),
        out_shape=jax.ShapeDtypeStruct((E, 16), F32),
    )(emb1, f2, ef2, aw1, ab1, aw2, ab2)


def _tc_feats(acc1, acc2, est, estp, conv1_w):
    g = N // BN
    gp = N // BN

    def body(a1a_ref, a1b_ref, a2a_ref, a2b_ref, es_ref, ep_ref, w_ref,
             hp_ref, dis_ref):
        ep = ep_ref[...]
        s1 = jnp.sum(ep[:, 0, 0])
        s2 = jnp.sum(ep[:, 0, 1])
        mean = s1 / N
        var = (s2 - N * mean * mean) / (N - 1)
        std = jnp.sqrt(jnp.maximum(var, 0.0))
        es = es_ref[0, 0] - mean
        es = jnp.where(std > 1e-8, es / jnp.where(std > 1e-8, std, 1.0), es)

        a1a = a1a_ref[...]
        a1b = a1b_ref[...]
        out_deg = a1a[:, 0]
        in_deg = a1b[:, 0]
        total = in_deg + out_deg
        nst = a1a[:, 1:4] + a1b[:, 1:4]
        a2 = a2a_ref[...] + a2b_ref[...]
        denom = a2[:, 0]
        vec = a2[:, 1:9]
        agg = vec / jnp.where(denom > 0, denom, 1.0)[:, None]
        feats = jnp.concatenate(
            [es[:, None], total[:, None], nst, agg], axis=1)
        h1 = jnp.dot(feats, w_ref[...], preferred_element_type=F32)
        dis = lax.rsqrt(in_deg + 1.0)
        hp = dis[:, None] * h1
        hp_ref[0] = hp[:, :32]
        hp_ref[1] = hp[:, 32:]
        dis_ref[0, 0] = dis

    return pl.pallas_call(
        body,
        grid=(g,),
        compiler_params=pltpu.CompilerParams(
            vmem_limit_bytes=60 * 1024 * 1024),
        in_specs=[
            pl.BlockSpec((BN, 8), lambda i: (i, 0)),
            pl.BlockSpec((BN, 8), lambda i: (i, 0)),
            pl.BlockSpec((BN, 16), lambda i: (i, 0)),
            pl.BlockSpec((BN, 16), lambda i: (i, 0)),
            pl.BlockSpec((1, 1, BN), lambda i: (i, 0, 0)),
            pl.BlockSpec((gp, 1, 8), lambda i: (0, 0, 0)),
            pl.BlockSpec((13, HID), lambda i: (0, 0)),
        ],
        out_specs=[
            pl.BlockSpec((2, BN, 32), lambda i: (0, i, 0)),
            pl.BlockSpec((1, 1, BN), lambda i: (i, 0, 0)),
        ],
        out_shape=[
            jax.ShapeDtypeStruct((2, N, 32), F32),
            jax.ShapeDtypeStruct((N // BN, 1, BN), F32),
        ],
    )(acc1[0], acc1[1], acc2[0], acc2[1], est, estp, conv1_w)


def _tc_layer(S, hp, dis, b_in, w_next):
    g = N // BN

    def body(sa_ref, hpa_ref, dis_ref, b_ref, w_ref, o_ref):
        sa = sa_ref[...]
        hpa = hpa_ref[...]
        dis = dis_ref[0, 0][:, None]
        pre = jnp.concatenate(
            [dis * (sa[0] + hpa[0]), dis * (sa[1] + hpa[1])], axis=1)
        x = jnp.maximum(pre + b_ref[...][None, :], 0.0)
        h = jnp.dot(x, w_ref[...], preferred_element_type=F32)
        hp = dis * h
        o_ref[0] = hp[:, :32]
        o_ref[1] = hp[:, 32:]

    return pl.pallas_call(
        body,
        grid=(g,),
        compiler_params=pltpu.CompilerParams(
            vmem_limit_bytes=60 * 1024 * 1024),
        in_specs=[
            pl.BlockSpec((2, BN, 32), lambda i: (0, i, 0)),
            pl.BlockSpec((2, BN, 32), lambda i: (0, i, 0)),
            pl.BlockSpec((1, 1, BN), lambda i: (i, 0, 0)),
            pl.BlockSpec((HID,), lambda i: (0,)),
            pl.BlockSpec((HID, HID), lambda i: (0, 0)),
        ],
        out_specs=pl.BlockSpec((2, BN, 32), lambda i: (0, i, 0)),
        out_shape=jax.ShapeDtypeStruct((2, N, 32), F32),
    )(S, hp, dis, b_in, w_next)


def _tc_x2(S, hp, dis, b_in, status):
    g = N // BN

    def body(sa_ref, hpa_ref, dis_ref, b_ref, st_ref, x_ref, p_ref):
        sa = sa_ref[...]
        hpa = hpa_ref[...]
        dis = dis_ref[0, 0][:, None]
        pre = jnp.concatenate(
            [dis * (sa[0] + hpa[0]), dis * (sa[1] + hpa[1])], axis=1)
        x = jnp.maximum(pre + b_ref[...][None, :], 0.0)
        x_ref[...] = x
        st = st_ref[0, 0]
        cnt1 = jnp.sum((st == 1).astype(F32))
        cnt0 = jnp.sum((st == 0).astype(F32))
        p = jnp.concatenate(
            [jnp.sum(x, axis=0), cnt1[None], cnt0[None],
             jnp.zeros((62,), F32)])
        p_ref[...] = p.reshape(1, 1, 128)

    return pl.pallas_call(
        body,
        grid=(g,),
        compiler_params=pltpu.CompilerParams(
            vmem_limit_bytes=60 * 1024 * 1024),
        in_specs=[
            pl.BlockSpec((2, BN, 32), lambda i: (0, i, 0)),
            pl.BlockSpec((2, BN, 32), lambda i: (0, i, 0)),
            pl.BlockSpec((1, 1, BN), lambda i: (i, 0, 0)),
            pl.BlockSpec((HID,), lambda i: (0,)),
            pl.BlockSpec((1, 1, BN), lambda i: (i, 0, 0)),
        ],
        out_specs=[
            pl.BlockSpec((BN, HID), lambda i: (i, 0)),
            pl.BlockSpec((1, 1, 128), lambda i: (i, 0, 0)),
        ],
        out_shape=[
            jax.ShapeDtypeStruct((N, HID), F32),
            jax.ShapeDtypeStruct((g, 1, 128), F32),
        ],
    )(S, hp, dis, b_in, status)


def _tc_heads(x2, partials, status, aw1, ab1, aw2, ab2):
    g = N // BN
    gp = N // BN

    def body(x_ref, p_ref, st_ref, aw1_ref, ab1_ref, aw2_ref, ab2_ref,
             lg_ref):
        p = jnp.sum(p_ref[...][:, 0, :], axis=0)
        gmean = (p[:64] / N).reshape(1, 64)
        cnt1 = p[64]
        x = x_ref[...]
        aw = aw1_ref[...]
        gterm = jnp.dot(gmean, aw[64:, :], preferred_element_type=F32) \
            + ab1_ref[...][None, :]
        h = jnp.maximum(
            jnp.dot(x, aw[:64, :], preferred_element_type=F32) + gterm, 0.0)
        raw = (jnp.dot(h, aw2_ref[...], preferred_element_type=F32)
               + ab2_ref[...][None, :])[:, 0]
        st = st_ref[0, 0]
        m1 = (st == 1).astype(F32)
        m0 = (st == 0).astype(F32)
        m = jnp.where(cnt1 > 0, m1, m0)
        lg_ref[0, 0] = raw * m + (1.0 - m) * (-1e9)

    return pl.pallas_call(
        body,
        grid=(g,),
        compiler_params=pltpu.CompilerParams(
            vmem_limit_bytes=60 * 1024 * 1024),
        in_specs=[
            pl.BlockSpec((BN, HID), lambda i: (i, 0)),
            pl.BlockSpec((gp, 1, 128), lambda i: (0, 0, 0)),
            pl.BlockSpec((1, 1, BN), lambda i: (i, 0, 0)),
            pl.BlockSpec((2 * HID, HID), lambda i: (0, 0)),
            pl.BlockSpec((HID,), lambda i: (0,)),
            pl.BlockSpec((HID, 1), lambda i: (0, 0)),
            pl.BlockSpec((1,), lambda i: (0,)),
        ],
        out_specs=pl.BlockSpec((1, 1, BN), lambda i: (i, 0, 0)),
        out_shape=jax.ShapeDtypeStruct((N // BN, 1, BN), F32),
    )(x2, partials, status, aw1, ab1, aw2, ab2)


def _mk_idxc(g2, s2):
    return jnp.concatenate(
        [g2.reshape(-1, _KA_NB, _KA_C), s2.reshape(-1, _KA_NB, _KA_C)],
        axis=1)


def kernel(status, est_size, edge_index, edge_feat1, edge_feat2, ef1_table,
           ef2_table, aw1, ab1, aw2, ab2, conv1_w, conv1_b, conv2_w, conv2_b,
           actor_w1, actor_b1, actor_w2, actor_b2,
           crit_w1, crit_b1, crit_w2, crit_b2, crit_w3, crit_b3):
    src = edge_index[0]
    dst = edge_index[1]
    st2 = status.astype(jnp.int32).reshape(N // BN, 1, BN)
    es2 = est_size.astype(F32).reshape(N // BN, 1, BN)

    nodeinfo, estp = _tc_nodeinfo(st2, es2)

    ef1p = jnp.pad(ef1_table.astype(F32), ((0, 0), (0, 3)))
    f1b = edge_feat1.astype(jnp.int32).reshape(32 * _K1_G, _K1_NB, _K1_C)
    emb1 = _k1_emb_gather(ef1p, f1b).reshape(E, 8)

    wemb = _tc_scores(emb1, ef2_table.astype(F32),
                      edge_feat2.astype(jnp.int32).reshape(E // BE, 1, BE),
                      aw1, ab1, aw2, ab2)

    src80 = src.astype(jnp.int32).reshape(10000, 80)
    dst80 = dst.astype(jnp.int32).reshape(10000, 80)
    z8 = jnp.zeros((N, 8), F32)
    acc1 = _ka_gather_scatter(
        nodeinfo,
        _mk_idxc(jnp.concatenate([dst80, src80], 0),
                 jnp.concatenate([src80, dst80], 0)),
        z8, 8).reshape(2, N, 8)

    src40 = src.astype(jnp.int32).reshape(-1, _KB_NB, _KB_C)
    dst40 = dst.astype(jnp.int32).reshape(-1, _KB_NB, _KB_C)
    z16 = jnp.zeros((N, 16), F32)
    acc2 = _kb_attn_scatter(
        wemb.reshape(20000, _KB_C, 16),
        jnp.concatenate([src40, dst40], axis=1), z16).reshape(2, N, 16)

    h1p, dis2 = _tc_feats(acc1, acc2, es2, estp, conv1_w)

    z32 = jnp.zeros((N, 32), F32)
    idxc5 = _mk_idxc(jnp.concatenate([src80, src80 + N], 0),
                     jnp.concatenate([dst80, dst80], 0))
    S1 = _ka_gather_scatter(h1p.reshape(2 * N, 32), idxc5,
                            z32, 32).reshape(2, N, 32)

    h2p = _tc_layer(S1, h1p, dis2, conv1_b, conv2_w)

    S2 = _ka_gather_scatter(h2p.reshape(2 * N, 32), idxc5,
                            z32, 32).reshape(2, N, 32)

    x2, partials = _tc_x2(S2, h2p, dis2, conv2_b, st2)

    logits = _tc_heads(x2, partials, st2,
                       actor_w1, actor_b1, actor_w2, actor_b2)

    gm = jnp.mean(x2, axis=0)
    combined = jnp.concatenate(
        [x2, jnp.broadcast_to(gm, (N, gm.shape[0]))], axis=1)
    pooled = jnp.mean(combined, axis=0)
    hv = jax.nn.relu(pooled @ crit_w1 + crit_b1)
    hv = jax.nn.relu(hv @ crit_w2 + crit_b2)
    value = (hv @ crit_w3 + crit_b3)[0]
    return (logits.reshape(N), value)
